# Initial kernel scaffold; baseline (speedup 1.0000x reference)
import functools

import jax
import jax.numpy as jnp
import numpy as np
from jax import lax
from jax.experimental import pallas as pl
from jax.experimental.pallas import tpu as pltpu
from jax.experimental.pallas import tpu_sc as plsc

VSHAPES = (1500, 750, 375, 188)
ASHAPES = (750, 375, 188, 94)
_B = 2
_D = 256
_H = 8
_P = 4
_L = 4
_DFF = 1024
_NL = 2
_KV = 600
_KA = 300
_DH = 32
_LV = int(sum(VSHAPES))
_LA = int(sum(ASHAPES))
_NVr = 5632
_NAr = 2816
_NW = 32

_f32 = jnp.float32
_i32 = jnp.int32


def _wid():
    return lax.axis_index("s") * 2 + lax.axis_index("c")


def _mesh():
    return plsc.VectorSubcoreMesh(core_axis_name="c", subcore_axis_name="s")


def _make_row_gather(n_idx_list, ch):
    nph = len(n_idx_list)
    out_type = [jax.ShapeDtypeStruct((n, _D), _f32) for n in n_idx_list]
    scratch = [pltpu.VMEM((ch,), _i32), pltpu.VMEM((ch, _D), _f32),
               pltpu.SemaphoreType.DMA]

    def body(*refs):
        tables = refs[:nph]
        idxs = refs[nph:2 * nph]
        outs = refs[2 * nph:3 * nph]
        idx_v, rows_v, sem = refs[3 * nph:]
        wid = _wid()
        for p in range(nph):
            n = n_idx_list[p]
            assert n % ch == 0
            nch = n // ch

            def one(c, _p=p):
                base = c * ch
                pltpu.sync_copy(idxs[_p].at[pl.ds(base, ch)], idx_v)
                pltpu.async_copy(tables[_p].at[idx_v], rows_v, sem).wait()
                pltpu.sync_copy(rows_v, outs[_p].at[pl.ds(base, ch)])

            if nch % _NW == 0:
                def bodyf(i, carry, _one=one):
                    _one(i * _NW + wid)
                    return carry
                lax.fori_loop(0, nch // _NW, bodyf, 0)
            else:
                def bodyf(i, carry, _one=one, _nch=nch):
                    c = i * _NW + wid

                    @pl.when(c < _nch)
                    def _():
                        _one(c)
                    return carry
                lax.fori_loop(0, -(-nch // _NW), bodyf, 0)

    return pl.kernel(body, out_type=out_type, mesh=_mesh(),
                     scratch_types=scratch)


def _make_deform(use_plut):
    ITV = _B * _KV * _H // _NW
    ITA = _B * _KA * _H // _NW
    CHV = 6
    CHA = 3
    NCH = ITV // CHV
    assert ITA // CHA == NCH

    out_type = [jax.ShapeDtypeStruct((_B * _KV * _H, _DH), _f32),
                jax.ShapeDtypeStruct((_B * _KA * _H, _DH), _f32)]
    scratch = [
        pltpu.VMEM((CHV * 32,), _i32),
        pltpu.VMEM((CHV * 32,), _f32),
        pltpu.VMEM((CHV * 32, _DH), _f32),
        pltpu.VMEM((CHV, _DH), _f32),
        pltpu.VMEM((CHA * 32,), _i32),
        pltpu.VMEM((CHA * 32,), _f32),
        pltpu.VMEM((CHA * 32, _DH), _f32),
        pltpu.VMEM((CHA, _DH), _f32),
        pltpu.SemaphoreType.DMA,
    ]
    if use_plut:
        scratch += [pltpu.VMEM((_NAr,), _i32), pltpu.VMEM((_NVr,), _i32)]

    def body(*refs):
        (tab_a, tab_v, rows_v, coef_v, rows_a, coef_a) = refs[:6]
        k = 6
        if use_plut:
            plut_a, plut_v = refs[6:8]
            k = 8
        out_v, out_a = refs[k:k + 2]
        (ixv, cbv, gv, obv, ixa, cba, ga, oba, sem) = refs[k + 2:k + 11]
        if use_plut:
            pav, pvv = refs[k + 11:k + 13]
            pltpu.sync_copy(plut_a, pav)
            pltpu.sync_copy(plut_v, pvv)
        wid = _wid()

        def side(rows, coef, tab, out, ipt, chi, ixb, cbb, gb, obb, plut_vm):
            ns = chi * 32

            def bodyf(c, carry):
                ibase = wid * ipt + c * chi
                sbase = ibase * 32
                pltpu.sync_copy(rows.at[pl.ds(sbase, ns)], ixb)
                pltpu.sync_copy(coef.at[pl.ds(sbase, ns)], cbb)
                if plut_vm is not None:
                    for g_ in range(ns // 16):
                        v = ixb[pl.ds(g_ * 16, 16)]
                        pos = lax.shift_right_logical(v, 3)
                        hh = lax.bitwise_and(v, 7)
                        m = plsc.load_gather(plut_vm, [pos])
                        ixb[pl.ds(g_ * 16, 16)] = m * _H + hh
                handles = []
                for s0 in range(0, ns, 96):
                    sl = min(96, ns - s0)
                    handles.append(pltpu.async_copy(
                        tab.at[ixb.at[pl.ds(s0, sl)]],
                        gb.at[pl.ds(s0, sl)], sem))
                for hd in handles:
                    hd.wait()
                for i in range(chi):
                    a0 = jnp.zeros((16,), _f32)
                    a1 = jnp.zeros((16,), _f32)
                    b0 = jnp.zeros((16,), _f32)
                    b1 = jnp.zeros((16,), _f32)
                    for s in range(32):
                        si = i * 32 + s
                        cv = plsc.load_gather(
                            cbb, [jnp.full((16,), si, _i32)])
                        if s % 2 == 0:
                            a0 = a0 + cv * gb[si, pl.ds(0, 16)]
                            a1 = a1 + cv * gb[si, pl.ds(16, 16)]
                        else:
                            b0 = b0 + cv * gb[si, pl.ds(0, 16)]
                            b1 = b1 + cv * gb[si, pl.ds(16, 16)]
                    obb[i, pl.ds(0, 16)] = a0 + b0
                    obb[i, pl.ds(16, 16)] = a1 + b1
                pltpu.sync_copy(obb, out.at[pl.ds(ibase, chi)])
                return carry

            lax.fori_loop(0, NCH, bodyf, 0)

        side(rows_v, coef_v, tab_a, out_v, ITV, CHV, ixv, cbv, gv, obv,
             pav if use_plut else None)
        side(rows_a, coef_a, tab_v, out_a, ITA, CHA, ixa, cba, ga, oba,
             pvv if use_plut else None)

    return pl.kernel(body, out_type=out_type, mesh=_mesh(),
                     scratch_types=scratch)


def _make_valproj(n_rows, rb):
    assert n_rows % rb == 0
    grid = (_NL, n_rows // rb)

    def body(x_ref, w_ref, b_ref, o_ref):
        o_ref[...] = (jnp.dot(x_ref[...], w_ref[0],
                              preferred_element_type=_f32) + b_ref[0])[None]

    return pl.pallas_call(
        body,
        grid=grid,
        in_specs=[
            pl.BlockSpec((rb, _D), lambda l, r: (r, 0)),
            pl.BlockSpec((1, _D, _D), lambda l, r: (l, 0, 0)),
            pl.BlockSpec((1, 1, _D), lambda l, r: (l, 0, 0)),
        ],
        out_specs=pl.BlockSpec((1, rb, _D), lambda l, r: (l, r, 0)),
        out_shape=jax.ShapeDtypeStruct((_NL, n_rows, _D), _f32),
    )


def _make_prep(n, k_per_b, lval):
    def body(tgt, pos, ref128, w4, b4, mm, llf, lsi, hvec,
             i0o, i1o, c0o, c1o):
        q = tgt[...] + pos[...]
        lg = jnp.dot(q, w4[...], preferred_element_type=_f32) + b4[...]
        brow = (lax.broadcasted_iota(_i32, (n, 128), 0) >= k_per_b
                ).astype(_i32)
        ref = ref128[...]
        ll = llf[...]
        st = lsi[...]
        hv = hvec[...]
        base = brow * lval + st
        for lid in range(_NL):
            off = lg[:, lid * 256:lid * 256 + 128]
            e = jnp.exp(lg[:, lid * 256 + 128:lid * 256 + 256])
            aw = e / jnp.dot(e, mm[...], preferred_element_type=_f32)
            x = ref * ll + off - 0.5
            x0 = jnp.floor(x)
            w = x - x0
            i0 = jnp.clip(x0, 0.0, ll - 1.0).astype(_i32)
            i1 = jnp.clip(x0 + 1.0, 0.0, ll - 1.0).astype(_i32)
            i0o[lid] = (base + i0) * _H + hv
            i1o[lid] = (base + i1) * _H + hv
            c0o[lid] = aw * (1.0 - w)
            c1o[lid] = aw * w

    return pl.pallas_call(
        body,
        out_shape=[
            jax.ShapeDtypeStruct((_NL, n, 128), _i32),
            jax.ShapeDtypeStruct((_NL, n, 128), _i32),
            jax.ShapeDtypeStruct((_NL, n, 128), _f32),
            jax.ShapeDtypeStruct((_NL, n, 128), _f32),
        ],
    )


def _ln(x, g, b):
    m = jnp.mean(x, -1, keepdims=True)
    v = jnp.mean((x - m) * (x - m), -1, keepdims=True)
    return (x - m) * lax.rsqrt(v + 1e-5) * g + b


def _make_layer(n, with_corr):
    def body(att, tgt, wout, bout, g1, b1, wf1, bf1, wf2, bf2, g2, b2,
             *rest):
        if with_corr:
            wvn, bvn, xo, co = rest
        else:
            (xo,) = rest
        t = jnp.dot(att[...], wout[...], preferred_element_type=_f32) \
            + bout[...]
        h1 = _ln(tgt[...] + t, g1[...], b1[...])
        f = jnp.dot(
            jnp.maximum(
                jnp.dot(h1, wf1[...], preferred_element_type=_f32)
                + bf1[...], 0.0),
            wf2[...], preferred_element_type=_f32) + bf2[...]
        x = _ln(h1 + f, g2[...], b2[...])
        xo[...] = x
        if with_corr:
            co[...] = jnp.dot(x, wvn[...], preferred_element_type=_f32) \
                + bvn[...]

    out_shape = [jax.ShapeDtypeStruct((n, _D), _f32)]
    if with_corr:
        out_shape.append(jax.ShapeDtypeStruct((n, _D), _f32))
    return pl.pallas_call(body, out_shape=out_shape)


_LANE = np.arange(128)
_L_OF_LANE = (_LANE % 16) // _P
_H_OF_LANE = _LANE // 16


def _lane_consts(shapes):
    ll = np.array(shapes, np.float32)[_L_OF_LANE][None]
    st = np.concatenate([[0], np.cumsum(shapes)[:-1]]).astype(np.int32)
    st = st[_L_OF_LANE][None]
    return jnp.asarray(ll), jnp.asarray(st)


def _ref_points(shapes, valid_ratios):
    refs = []
    for l, ll in enumerate(shapes):
        r = (jnp.arange(ll, dtype=_f32) + 0.5)
        r = r[None, :] / (valid_ratios[:, l:l + 1] * ll)
        refs.append(r)
    ref = jnp.concatenate(refs, 1)
    return ref[:, :, None] * valid_ratios[:, None, :]


def _to_flat(a0, a1):
    nl, n, _ = a0.shape
    a = a0.reshape(nl, n, _H, 16)
    b = a1.reshape(nl, n, _H, 16)
    return jnp.concatenate([a, b], axis=-1).reshape(nl, n * _H * 32)


_K = {}


def _build():
    if _K:
        return
    _K["gather0"] = _make_row_gather(
        [_B * _KV, _B * _KV, _B * _KA, _B * _KA], 40)
    _K["finalg"] = _make_row_gather([_B * (_LV + _LA)], 40)
    _K["deform0"] = _make_deform(False)
    _K["deform1"] = _make_deform(True)
    _K["valproj_v"] = _make_valproj(_NVr, 512)
    _K["valproj_a"] = _make_valproj(_NAr, 512)
    _K["prep_v"] = _make_prep(_B * _KV, _KV, _LA)
    _K["prep_a"] = _make_prep(_B * _KA, _KA, _LV)
    _K["layer_v_c"] = _make_layer(_B * _KV, True)
    _K["layer_v"] = _make_layer(_B * _KV, False)
    _K["layer_a_c"] = _make_layer(_B * _KA, True)
    _K["layer_a"] = _make_layer(_B * _KA, False)


def kernel(video_src, video_pos, video_topk, video_valid_ratios,
           audio_src, audio_pos, audio_topk, audio_valid_ratios,
           W_off, b_off, W_attn, b_attn, W_val, b_val, W_out, b_out,
           ln1_g, ln1_b, ln2_g, ln2_b, W_ff1, b_ff1, W_ff2, b_ff2):
    _build()
    bi = jnp.arange(_B, dtype=_i32)[:, None]
    topk_v = video_topk.astype(_i32)
    topk_a = audio_topk.astype(_i32)
    gidx_v = (bi * _LV + topk_v).reshape(-1)
    gidx_a = (bi * _LA + topk_a).reshape(-1)
    srcv = video_src.reshape(_B * _LV, _D)
    posv = video_pos.reshape(_B * _LV, _D)
    srca = audio_src.reshape(_B * _LA, _D)
    posa = audio_pos.reshape(_B * _LA, _D)

    tgt_v, pos_v_t, tgt_a, pos_a_t = _K["gather0"](
        srcv, posv, srca, posa, gidx_v, gidx_v, gidx_a, gidx_a)

    srcv_p = jnp.pad(srcv, ((0, _NVr - _B * _LV), (0, 0)))
    srca_p = jnp.pad(srca, ((0, _NAr - _B * _LA), (0, 0)))
    vbase_v = _K["valproj_v"](srcv_p, W_val[:, 1],
                              b_val[:, 1].reshape(_NL, 1, _D))
    vbase_a = _K["valproj_a"](srca_p, W_val[:, 0],
                              b_val[:, 0].reshape(_NL, 1, _D))

    ref_v = _ref_points(VSHAPES, video_valid_ratios)
    ref_a = _ref_points(ASHAPES, audio_valid_ratios)
    ref_v_t = ref_v[bi, topk_v].reshape(_B * _KV, _L)
    ref_a_t = ref_a[bi, topk_a].reshape(_B * _KA, _L)
    ref128_v = jnp.tile(jnp.repeat(ref_v_t, _P, axis=1), (1, _H))
    ref128_a = jnp.tile(jnp.repeat(ref_a_t, _P, axis=1), (1, _H))

    mm = jnp.asarray(np.kron(np.eye(_H, dtype=np.float32),
                             np.ones((16, 16), np.float32)))
    llf_a, lsi_a = _lane_consts(ASHAPES)
    llf_v, lsi_v = _lane_consts(VSHAPES)
    hvec = jnp.asarray(_H_OF_LANE.astype(np.int32))[None]

    def _w4(m):
        return (jnp.concatenate([W_off[0, m], W_attn[0, m],
                                 W_off[1, m], W_attn[1, m]], axis=1),
                jnp.concatenate([b_off[0, m], b_attn[0, m],
                                 b_off[1, m], b_attn[1, m]])[None])

    w4v, b4v = _w4(0)
    w4a, b4a = _w4(1)
    i0v, i1v, c0v, c1v = _K["prep_v"](tgt_v, pos_v_t, ref128_v, w4v, b4v,
                                      mm, llf_a, lsi_a, hvec)
    i0a, i1a, c0a, c1a = _K["prep_a"](tgt_a, pos_a_t, ref128_a, w4a, b4a,
                                      mm, llf_v, lsi_v, hvec)
    rows_v = _to_flat(i0v, i1v)
    coef_v = _to_flat(c0v, c1v)
    rows_a = _to_flat(i0a, i1a)
    coef_a = _to_flat(c0a, c1a)

    rankv = jnp.broadcast_to(jnp.arange(_KV, dtype=_i32)[None], (_B, _KV))
    ranka = jnp.broadcast_to(jnp.arange(_KA, dtype=_i32)[None], (_B, _KA))
    corrpos_v = (bi * _KV + rankv).reshape(-1)
    corrpos_a = (bi * _KA + ranka).reshape(-1)
    plut_v1 = jnp.arange(_NVr, dtype=_i32).at[gidx_v].set(_NVr + corrpos_v)
    plut_a1 = jnp.arange(_NAr, dtype=_i32).at[gidx_a].set(_NAr + corrpos_a)

    tab_a0 = vbase_a[0].reshape(_NAr * _H, _DH)
    tab_v0 = vbase_v[0].reshape(_NVr * _H, _DH)
    av0, aa0 = _K["deform0"](tab_a0, tab_v0, rows_v[0], coef_v[0],
                             rows_a[0], coef_a[0])
    x_v0, corr_v0 = _K["layer_v_c"](
        av0.reshape(_B * _KV, _D), tgt_v, W_out[0, 0],
        b_out[0, 0][None], ln1_g[0, 0][None], ln1_b[0, 0][None],
        W_ff1[0, 0], b_ff1[0, 0][None], W_ff2[0, 0], b_ff2[0, 0][None],
        ln2_g[0, 0][None], ln2_b[0, 0][None], W_val[1, 1],
        b_val[1, 1][None])
    x_a0, corr_a0 = _K["layer_a_c"](
        aa0.reshape(_B * _KA, _D), tgt_a, W_out[0, 1],
        b_out[0, 1][None], ln1_g[0, 1][None], ln1_b[0, 1][None],
        W_ff1[0, 1], b_ff1[0, 1][None], W_ff2[0, 1], b_ff2[0, 1][None],
        ln2_g[0, 1][None], ln2_b[0, 1][None], W_val[1, 0],
        b_val[1, 0][None])

    tab_v1 = jnp.concatenate([vbase_v[1].reshape(_NVr * _H, _DH),
                              corr_v0.reshape(_B * _KV * _H, _DH)], 0)
    tab_a1 = jnp.concatenate([vbase_a[1].reshape(_NAr * _H, _DH),
                              corr_a0.reshape(_B * _KA * _H, _DH)], 0)
    av1, aa1 = _K["deform1"](tab_a1, tab_v1, rows_v[1], coef_v[1],
                             rows_a[1], coef_a[1], plut_a1, plut_v1)
    (x_v1,) = _K["layer_v"](
        av1.reshape(_B * _KV, _D), tgt_v, W_out[1, 0],
        b_out[1, 0][None], ln1_g[1, 0][None], ln1_b[1, 0][None],
        W_ff1[1, 0], b_ff1[1, 0][None], W_ff2[1, 0], b_ff2[1, 0][None],
        ln2_g[1, 0][None], ln2_b[1, 0][None])
    (x_a1,) = _K["layer_a"](
        aa1.reshape(_B * _KA, _D), tgt_a, W_out[1, 1],
        b_out[1, 1][None], ln1_g[1, 1][None], ln1_b[1, 1][None],
        W_ff1[1, 1], b_ff1[1, 1][None], W_ff2[1, 1], b_ff2[1, 1][None],
        ln2_g[1, 1][None], ln2_b[1, 1][None])

    ftab = jnp.concatenate([srcv, x_v1, srca, x_a1], 0)
    nv, na = _B * _LV, _B * _LA
    lv = jnp.arange(nv, dtype=_i32).at[gidx_v].set(nv + corrpos_v)
    la = (nv + _B * _KV + jnp.arange(na, dtype=_i32)).at[gidx_a].set(
        nv + _B * _KV + na + corrpos_a)
    lut_full = jnp.concatenate([lv[:_LV], la[:_LA],
                                lv[_LV:], la[_LA:]], 0)
    (out_flat,) = _K["finalg"](ftab, lut_full)
    return out_flat.reshape(_B, _LV + _LA, _D)


# speedup vs baseline: 10.3211x; 10.3211x over previous
import functools

import jax
import jax.numpy as jnp
import numpy as np
from jax import lax
from jax.experimental import pallas as pl
from jax.experimental.pallas import tpu as pltpu
from jax.experimental.pallas import tpu_sc as plsc

VSHAPES = (1500, 750, 375, 188)
ASHAPES = (750, 375, 188, 94)
_B = 2
_D = 256
_H = 8
_P = 4
_L = 4
_DFF = 1024
_NL = 2
_KV = 600
_KA = 300
_DH = 32
_LV = int(sum(VSHAPES))
_LA = int(sum(ASHAPES))
_NVr = 5632
_NAr = 2816
_NW = 32

_f32 = jnp.float32
_i32 = jnp.int32
_SC_PARAMS = pltpu.CompilerParams(use_tc_tiling_on_sc=False,
                                  needs_layout_passes=False)


def _wid():
    return lax.axis_index("s") * 2 + lax.axis_index("c")


def _mesh():
    return plsc.VectorSubcoreMesh(core_axis_name="c", subcore_axis_name="s")


def _make_row_gather(n_idx_list, ch):
    nph = len(n_idx_list)
    out_type = [jax.ShapeDtypeStruct((n, _D), _f32) for n in n_idx_list]
    scratch = [pltpu.VMEM((ch,), _i32), pltpu.VMEM((ch, _D), _f32),
               pltpu.SemaphoreType.DMA]

    def body(*refs):
        tables = refs[:nph]
        idxs = refs[nph:2 * nph]
        outs = refs[2 * nph:3 * nph]
        idx_v, rows_v, sem = refs[3 * nph:]
        wid = _wid()
        for p in range(nph):
            n = n_idx_list[p]
            assert n % ch == 0
            nch = n // ch

            def one(c, _p=p):
                base = c * ch
                pltpu.sync_copy(idxs[_p].at[pl.ds(base, ch)], idx_v)
                pltpu.async_copy(tables[_p].at[idx_v], rows_v, sem).wait()
                pltpu.sync_copy(rows_v, outs[_p].at[pl.ds(base, ch)])

            if nch % _NW == 0:
                def bodyf(i, carry, _one=one):
                    _one(i * _NW + wid)
                    return carry
                lax.fori_loop(0, nch // _NW, bodyf, 0)
            else:
                def bodyf(i, carry, _one=one, _nch=nch):
                    c = i * _NW + wid

                    @pl.when(c < _nch)
                    def _():
                        _one(c)
                    return carry
                lax.fori_loop(0, -(-nch // _NW), bodyf, 0)

    return pl.kernel(body, out_type=out_type, mesh=_mesh(),
                     scratch_types=scratch, compiler_params=_SC_PARAMS)


def _make_deform(use_plut):
    ITV = _B * _KV * _H // _NW
    ITA = _B * _KA * _H // _NW
    CHV = 6
    CHA = 3
    NCH = ITV // CHV
    assert ITA // CHA == NCH

    out_type = [jax.ShapeDtypeStruct((_B * _KV * _H, _DH), _f32),
                jax.ShapeDtypeStruct((_B * _KA * _H, _DH), _f32)]
    scratch = [
        pltpu.VMEM((CHV * 32,), _i32),
        pltpu.VMEM((CHV * 32,), _f32),
        pltpu.VMEM((CHV * 32, _DH), _f32),
        pltpu.VMEM((CHV, _DH), _f32),
        pltpu.VMEM((CHA * 32,), _i32),
        pltpu.VMEM((CHA * 32,), _f32),
        pltpu.VMEM((CHA * 32, _DH), _f32),
        pltpu.VMEM((CHA, _DH), _f32),
        pltpu.SemaphoreType.DMA,
    ]
    if use_plut:
        scratch += [pltpu.VMEM((_NAr,), _i32), pltpu.VMEM((_NVr,), _i32)]

    def body(*refs):
        (tab_a, tab_v, rows_v, coef_v, rows_a, coef_a) = refs[:6]
        k = 6
        if use_plut:
            plut_a, plut_v = refs[6:8]
            k = 8
        out_v, out_a = refs[k:k + 2]
        (ixv, cbv, gv, obv, ixa, cba, ga, oba, sem) = refs[k + 2:k + 11]
        if use_plut:
            pav, pvv = refs[k + 11:k + 13]
            pltpu.sync_copy(plut_a, pav)
            pltpu.sync_copy(plut_v, pvv)
        wid = _wid()

        def side(rows, coef, tab, out, ipt, chi, ixb, cbb, gb, obb, plut_vm):
            ns = chi * 32

            def bodyf(c, carry):
                ibase = wid * ipt + c * chi
                sbase = ibase * 32
                pltpu.sync_copy(rows.at[pl.ds(sbase, ns)], ixb)
                pltpu.sync_copy(coef.at[pl.ds(sbase, ns)], cbb)
                if plut_vm is not None:
                    for g_ in range(ns // 16):
                        v = ixb[pl.ds(g_ * 16, 16)]
                        pos = lax.shift_right_logical(v, 3)
                        hh = lax.bitwise_and(v, 7)
                        m = plsc.load_gather(plut_vm, [pos])
                        ixb[pl.ds(g_ * 16, 16)] = m * _H + hh
                handles = []
                for s0 in range(0, ns, 96):
                    sl = min(96, ns - s0)
                    handles.append(pltpu.async_copy(
                        tab.at[ixb.at[pl.ds(s0, sl)]],
                        gb.at[pl.ds(s0, sl)], sem))
                for hd in handles:
                    hd.wait()
                for i in range(chi):
                    a0 = jnp.zeros((16,), _f32)
                    a1 = jnp.zeros((16,), _f32)
                    b0 = jnp.zeros((16,), _f32)
                    b1 = jnp.zeros((16,), _f32)
                    for s in range(32):
                        si = i * 32 + s
                        cv = plsc.load_gather(
                            cbb, [jnp.full((16,), si, _i32)])
                        if s % 2 == 0:
                            a0 = a0 + cv * gb[si, pl.ds(0, 16)]
                            a1 = a1 + cv * gb[si, pl.ds(16, 16)]
                        else:
                            b0 = b0 + cv * gb[si, pl.ds(0, 16)]
                            b1 = b1 + cv * gb[si, pl.ds(16, 16)]
                    obb[i, pl.ds(0, 16)] = a0 + b0
                    obb[i, pl.ds(16, 16)] = a1 + b1
                pltpu.sync_copy(obb, out.at[pl.ds(ibase, chi)])
                return carry

            lax.fori_loop(0, NCH, bodyf, 0)

        side(rows_v, coef_v, tab_a, out_v, ITV, CHV, ixv, cbv, gv, obv,
             pav if use_plut else None)
        side(rows_a, coef_a, tab_v, out_a, ITA, CHA, ixa, cba, ga, oba,
             pvv if use_plut else None)

    return pl.kernel(body, out_type=out_type, mesh=_mesh(),
                     scratch_types=scratch, compiler_params=_SC_PARAMS)


def _make_valproj(n_rows, rb):
    assert n_rows % rb == 0
    grid = (_NL, n_rows // rb)

    def body(x_ref, w_ref, b_ref, o_ref):
        o_ref[...] = (jnp.dot(x_ref[...], w_ref[0],
                              preferred_element_type=_f32) + b_ref[0])[None]

    return pl.pallas_call(
        body,
        grid=grid,
        in_specs=[
            pl.BlockSpec((rb, _D), lambda l, r: (r, 0)),
            pl.BlockSpec((1, _D, _D), lambda l, r: (l, 0, 0)),
            pl.BlockSpec((1, 1, _D), lambda l, r: (l, 0, 0)),
        ],
        out_specs=pl.BlockSpec((1, rb, _D), lambda l, r: (l, r, 0)),
        out_shape=jax.ShapeDtypeStruct((_NL, n_rows, _D), _f32),
    )


def _make_prep(n, k_per_b, lval):
    def body(tgt, pos, ref128, w4, b4, mm, llf, lsi, hvec,
             i0o, i1o, c0o, c1o):
        q = tgt[...] + pos[...]
        lg = jnp.dot(q, w4[...], preferred_element_type=_f32) + b4[...]
        brow = (lax.broadcasted_iota(_i32, (n, 128), 0) >= k_per_b
                ).astype(_i32)
        ref = ref128[...]
        ll = llf[...]
        st = lsi[...]
        hv = hvec[...]
        base = brow * lval + st
        for lid in range(_NL):
            off = lg[:, lid * 256:lid * 256 + 128]
            e = jnp.exp(lg[:, lid * 256 + 128:lid * 256 + 256])
            aw = e / jnp.dot(e, mm[...], preferred_element_type=_f32)
            x = ref * ll + off - 0.5
            x0 = jnp.floor(x)
            w = x - x0
            i0 = jnp.clip(x0, 0.0, ll - 1.0).astype(_i32)
            i1 = jnp.clip(x0 + 1.0, 0.0, ll - 1.0).astype(_i32)
            i0o[lid] = (base + i0) * _H + hv
            i1o[lid] = (base + i1) * _H + hv
            c0o[lid] = aw * (1.0 - w)
            c1o[lid] = aw * w

    return pl.pallas_call(
        body,
        out_shape=[
            jax.ShapeDtypeStruct((_NL, n, 128), _i32),
            jax.ShapeDtypeStruct((_NL, n, 128), _i32),
            jax.ShapeDtypeStruct((_NL, n, 128), _f32),
            jax.ShapeDtypeStruct((_NL, n, 128), _f32),
        ],
    )


def _ln(x, g, b):
    m = jnp.mean(x, -1, keepdims=True)
    v = jnp.mean((x - m) * (x - m), -1, keepdims=True)
    return (x - m) * lax.rsqrt(v + 1e-5) * g + b


def _make_layer(n, with_corr):
    def body(att, tgt, wout, bout, g1, b1, wf1, bf1, wf2, bf2, g2, b2,
             *rest):
        if with_corr:
            wvn, bvn, xo, co = rest
        else:
            (xo,) = rest
        t = jnp.dot(att[...], wout[...], preferred_element_type=_f32) \
            + bout[...]
        h1 = _ln(tgt[...] + t, g1[...], b1[...])
        f = jnp.dot(
            jnp.maximum(
                jnp.dot(h1, wf1[...], preferred_element_type=_f32)
                + bf1[...], 0.0),
            wf2[...], preferred_element_type=_f32) + bf2[...]
        x = _ln(h1 + f, g2[...], b2[...])
        xo[...] = x
        if with_corr:
            co[...] = jnp.dot(x, wvn[...], preferred_element_type=_f32) \
                + bvn[...]

    out_shape = [jax.ShapeDtypeStruct((n, _D), _f32)]
    if with_corr:
        out_shape.append(jax.ShapeDtypeStruct((n, _D), _f32))
    return pl.pallas_call(body, out_shape=out_shape)


_LANE = np.arange(128)
_L_OF_LANE = (_LANE % 16) // _P
_H_OF_LANE = _LANE // 16


def _lane_consts(shapes):
    ll = np.array(shapes, np.float32)[_L_OF_LANE][None]
    st = np.concatenate([[0], np.cumsum(shapes)[:-1]]).astype(np.int32)
    st = st[_L_OF_LANE][None]
    return jnp.asarray(ll), jnp.asarray(st)


def _ref_points(shapes, valid_ratios):
    refs = []
    for l, ll in enumerate(shapes):
        r = (jnp.arange(ll, dtype=_f32) + 0.5)
        r = r[None, :] / (valid_ratios[:, l:l + 1] * ll)
        refs.append(r)
    ref = jnp.concatenate(refs, 1)
    return ref[:, :, None] * valid_ratios[:, None, :]


def _to_flat(a0, a1):
    nl, n, _ = a0.shape
    a = a0.reshape(nl, n, _H, 16)
    b = a1.reshape(nl, n, _H, 16)
    return jnp.concatenate([a, b], axis=-1).reshape(nl, n * _H * 32)


_K = {}


def _build():
    if _K:
        return
    _K["gather0"] = _make_row_gather(
        [_B * _KV, _B * _KV, _B * _KA, _B * _KA], 40)
    _K["finalg"] = _make_row_gather([_B * (_LV + _LA)], 40)
    _K["deform0"] = _make_deform(False)
    _K["deform1"] = _make_deform(True)
    _K["valproj_v"] = _make_valproj(_NVr, 512)
    _K["valproj_a"] = _make_valproj(_NAr, 256)
    _K["prep_v"] = _make_prep(_B * _KV, _KV, _LA)
    _K["prep_a"] = _make_prep(_B * _KA, _KA, _LV)
    _K["layer_v_c"] = _make_layer(_B * _KV, True)
    _K["layer_v"] = _make_layer(_B * _KV, False)
    _K["layer_a_c"] = _make_layer(_B * _KA, True)
    _K["layer_a"] = _make_layer(_B * _KA, False)


def kernel(video_src, video_pos, video_topk, video_valid_ratios,
           audio_src, audio_pos, audio_topk, audio_valid_ratios,
           W_off, b_off, W_attn, b_attn, W_val, b_val, W_out, b_out,
           ln1_g, ln1_b, ln2_g, ln2_b, W_ff1, b_ff1, W_ff2, b_ff2):
    _build()
    bi = jnp.arange(_B, dtype=_i32)[:, None]
    topk_v = video_topk.astype(_i32)
    topk_a = audio_topk.astype(_i32)
    gidx_v = (bi * _LV + topk_v).reshape(-1)
    gidx_a = (bi * _LA + topk_a).reshape(-1)
    srcv = video_src.reshape(_B * _LV, _D)
    posv = video_pos.reshape(_B * _LV, _D)
    srca = audio_src.reshape(_B * _LA, _D)
    posa = audio_pos.reshape(_B * _LA, _D)

    tgt_v, pos_v_t, tgt_a, pos_a_t = _K["gather0"](
        srcv, posv, srca, posa, gidx_v, gidx_v, gidx_a, gidx_a)

    srcv_p = jnp.pad(srcv, ((0, _NVr - _B * _LV), (0, 0)))
    srca_p = jnp.pad(srca, ((0, _NAr - _B * _LA), (0, 0)))
    vbase_v = _K["valproj_v"](srcv_p, W_val[:, 1],
                              b_val[:, 1].reshape(_NL, 1, _D))
    vbase_a = _K["valproj_a"](srca_p, W_val[:, 0],
                              b_val[:, 0].reshape(_NL, 1, _D))

    ref_v = _ref_points(VSHAPES, video_valid_ratios)
    ref_a = _ref_points(ASHAPES, audio_valid_ratios)
    ref_v_t = ref_v[bi, topk_v].reshape(_B * _KV, _L)
    ref_a_t = ref_a[bi, topk_a].reshape(_B * _KA, _L)
    ref128_v = jnp.tile(jnp.repeat(ref_v_t, _P, axis=1), (1, _H))
    ref128_a = jnp.tile(jnp.repeat(ref_a_t, _P, axis=1), (1, _H))

    mm = jnp.asarray(np.kron(np.eye(_H, dtype=np.float32),
                             np.ones((16, 16), np.float32)))
    llf_a, lsi_a = _lane_consts(ASHAPES)
    llf_v, lsi_v = _lane_consts(VSHAPES)
    hvec = jnp.asarray(_H_OF_LANE.astype(np.int32))[None]

    def _w4(m):
        return (jnp.concatenate([W_off[0, m], W_attn[0, m],
                                 W_off[1, m], W_attn[1, m]], axis=1),
                jnp.concatenate([b_off[0, m], b_attn[0, m],
                                 b_off[1, m], b_attn[1, m]])[None])

    w4v, b4v = _w4(0)
    w4a, b4a = _w4(1)
    i0v, i1v, c0v, c1v = _K["prep_v"](tgt_v, pos_v_t, ref128_v, w4v, b4v,
                                      mm, llf_a, lsi_a, hvec)
    i0a, i1a, c0a, c1a = _K["prep_a"](tgt_a, pos_a_t, ref128_a, w4a, b4a,
                                      mm, llf_v, lsi_v, hvec)
    rows_v = _to_flat(i0v, i1v)
    coef_v = _to_flat(c0v, c1v)
    rows_a = _to_flat(i0a, i1a)
    coef_a = _to_flat(c0a, c1a)

    rankv = jnp.broadcast_to(jnp.arange(_KV, dtype=_i32)[None], (_B, _KV))
    ranka = jnp.broadcast_to(jnp.arange(_KA, dtype=_i32)[None], (_B, _KA))
    corrpos_v = (bi * _KV + rankv).reshape(-1)
    corrpos_a = (bi * _KA + ranka).reshape(-1)
    plut_v1 = jnp.arange(_NVr, dtype=_i32).at[gidx_v].set(_NVr + corrpos_v)
    plut_a1 = jnp.arange(_NAr, dtype=_i32).at[gidx_a].set(_NAr + corrpos_a)

    tab_a0 = vbase_a[0].reshape(_NAr * _H, _DH)
    tab_v0 = vbase_v[0].reshape(_NVr * _H, _DH)
    av0, aa0 = _K["deform0"](tab_a0, tab_v0, rows_v[0], coef_v[0],
                             rows_a[0], coef_a[0])
    x_v0, corr_v0 = _K["layer_v_c"](
        av0.reshape(_B * _KV, _D), tgt_v, W_out[0, 0],
        b_out[0, 0][None], ln1_g[0, 0][None], ln1_b[0, 0][None],
        W_ff1[0, 0], b_ff1[0, 0][None], W_ff2[0, 0], b_ff2[0, 0][None],
        ln2_g[0, 0][None], ln2_b[0, 0][None], W_val[1, 1],
        b_val[1, 1][None])
    x_a0, corr_a0 = _K["layer_a_c"](
        aa0.reshape(_B * _KA, _D), tgt_a, W_out[0, 1],
        b_out[0, 1][None], ln1_g[0, 1][None], ln1_b[0, 1][None],
        W_ff1[0, 1], b_ff1[0, 1][None], W_ff2[0, 1], b_ff2[0, 1][None],
        ln2_g[0, 1][None], ln2_b[0, 1][None], W_val[1, 0],
        b_val[1, 0][None])

    tab_v1 = jnp.concatenate([vbase_v[1].reshape(_NVr * _H, _DH),
                              corr_v0.reshape(_B * _KV * _H, _DH)], 0)
    tab_a1 = jnp.concatenate([vbase_a[1].reshape(_NAr * _H, _DH),
                              corr_a0.reshape(_B * _KA * _H, _DH)], 0)
    av1, aa1 = _K["deform1"](tab_a1, tab_v1, rows_v[1], coef_v[1],
                             rows_a[1], coef_a[1], plut_a1, plut_v1)
    (x_v1,) = _K["layer_v"](
        av1.reshape(_B * _KV, _D), tgt_v, W_out[1, 0],
        b_out[1, 0][None], ln1_g[1, 0][None], ln1_b[1, 0][None],
        W_ff1[1, 0], b_ff1[1, 0][None], W_ff2[1, 0], b_ff2[1, 0][None],
        ln2_g[1, 0][None], ln2_b[1, 0][None])
    (x_a1,) = _K["layer_a"](
        aa1.reshape(_B * _KA, _D), tgt_a, W_out[1, 1],
        b_out[1, 1][None], ln1_g[1, 1][None], ln1_b[1, 1][None],
        W_ff1[1, 1], b_ff1[1, 1][None], W_ff2[1, 1], b_ff2[1, 1][None],
        ln2_g[1, 1][None], ln2_b[1, 1][None])

    ftab = jnp.concatenate([srcv, x_v1, srca, x_a1], 0)
    nv, na = _B * _LV, _B * _LA
    lv = jnp.arange(nv, dtype=_i32).at[gidx_v].set(nv + corrpos_v)
    la = (nv + _B * _KV + jnp.arange(na, dtype=_i32)).at[gidx_a].set(
        nv + _B * _KV + na + corrpos_a)
    lut_full = jnp.concatenate([lv[:_LV], la[:_LA],
                                lv[_LV:], la[_LA:]], 0)
    (out_flat,) = _K["finalg"](ftab, lut_full)
    return out_flat.reshape(_B, _LV + _LA, _D)


# speedup vs baseline: 14.5797x; 1.4126x over previous
import functools

import jax
import jax.numpy as jnp
import numpy as np
from jax import lax
from jax.experimental import pallas as pl
from jax.experimental.pallas import tpu as pltpu
from jax.experimental.pallas import tpu_sc as plsc

VSHAPES = (1500, 750, 375, 188)
ASHAPES = (750, 375, 188, 94)
_B = 2
_D = 256
_H = 8
_P = 4
_L = 4
_DFF = 1024
_NL = 2
_KV = 600
_KA = 300
_DH = 32
_LV = int(sum(VSHAPES))
_LA = int(sum(ASHAPES))
_NVr = 5632
_NAr = 2816
_NW = 32

_f32 = jnp.float32
_i32 = jnp.int32
_SC_PARAMS = pltpu.CompilerParams(use_tc_tiling_on_sc=False,
                                  needs_layout_passes=False)


def _wid():
    return lax.axis_index("s") * 2 + lax.axis_index("c")


def _mesh():
    return plsc.VectorSubcoreMesh(core_axis_name="c", subcore_axis_name="s")


def _make_row_gather(n_idx_list, ch):
    nph = len(n_idx_list)
    out_type = [jax.ShapeDtypeStruct((n, _D), _f32) for n in n_idx_list]
    scratch = [pltpu.VMEM((ch,), _i32), pltpu.VMEM((ch, _D), _f32),
               pltpu.SemaphoreType.DMA]

    def body(*refs):
        tables = refs[:nph]
        idxs = refs[nph:2 * nph]
        outs = refs[2 * nph:3 * nph]
        idx_v, rows_v, sem = refs[3 * nph:]
        wid = _wid()
        for p in range(nph):
            n = n_idx_list[p]
            assert n % ch == 0
            nch = n // ch

            def one(c, _p=p):
                base = c * ch
                pltpu.sync_copy(idxs[_p].at[pl.ds(base, ch)], idx_v)
                pltpu.async_copy(tables[_p].at[idx_v], rows_v, sem).wait()
                pltpu.sync_copy(rows_v, outs[_p].at[pl.ds(base, ch)])

            if nch % _NW == 0:
                def bodyf(i, carry, _one=one):
                    _one(i * _NW + wid)
                    return carry
                lax.fori_loop(0, nch // _NW, bodyf, 0)
            else:
                def bodyf(i, carry, _one=one, _nch=nch):
                    c = i * _NW + wid

                    @pl.when(c < _nch)
                    def _():
                        _one(c)
                    return carry
                lax.fori_loop(0, -(-nch // _NW), bodyf, 0)

    return pl.kernel(body, out_type=out_type, mesh=_mesh(),
                     scratch_types=scratch, compiler_params=_SC_PARAMS)


def _make_deform(use_plut):
    ITV = _B * _KV * _H // _NW
    ITA = _B * _KA * _H // _NW
    CHV = 10
    CHA = 5
    NCH = ITV // CHV
    assert ITA // CHA == NCH and NCH % 2 == 0
    SV = ITV * 32
    NSV = CHV * 32

    out_type = [jax.ShapeDtypeStruct((_B * _KV * _H * _DH,), _f32),
                jax.ShapeDtypeStruct((_B * _KA * _H * _DH,), _f32)]
    scratch = [
        pltpu.VMEM((SV,), _i32),
        pltpu.VMEM((SV,), _f32),
        pltpu.VMEM((SV,), _f32),
        pltpu.VMEM((NSV, _DH), _f32),
        pltpu.VMEM((NSV, _DH), _f32),
        pltpu.SemaphoreType.DMA,
        pltpu.SemaphoreType.DMA,
    ]
    if use_plut:
        scratch += [pltpu.VMEM((_NAr,), _i32), pltpu.VMEM((_NVr,), _i32)]

    def body(*refs):
        (tab_a, tab_v, rows_v, coef_v, rows_a, coef_a) = refs[:6]
        k = 6
        if use_plut:
            plut_a, plut_v = refs[6:8]
            k = 8
        out_v, out_a = refs[k:k + 2]
        (ix, cb, ob, g0, g1, sem0, sem1) = refs[k + 2:k + 9]
        if use_plut:
            pav, pvv = refs[k + 9:k + 11]
            pltpu.sync_copy(plut_a, pav)
            pltpu.sync_copy(plut_v, pvv)
        wid = _wid()

        def side(rows, coef, tab, out, ipt, chi, plut_vm):
            slab = ipt * 32
            base = wid * slab
            ns = chi * 32
            pltpu.sync_copy(rows.at[pl.ds(base, slab)], ix.at[pl.ds(0, slab)])
            pltpu.sync_copy(coef.at[pl.ds(base, slab)], cb.at[pl.ds(0, slab)])
            if plut_vm is not None:
                def remap(g, carry):
                    v = ix[pl.ds(g * 16, 16)]
                    pos = lax.shift_right_logical(v, 3)
                    hh = lax.bitwise_and(v, 7)
                    m = plsc.load_gather(plut_vm, [pos])
                    ix[pl.ds(g * 16, 16)] = m * _H + hh
                    return carry
                lax.fori_loop(0, slab // 16, remap, 0)

            def fire(c, gb, sem):
                off = c * ns
                for s0 in range(0, ns, 96):
                    sl = min(96, ns - s0)
                    pltpu.async_copy(tab.at[ix.at[pl.ds(off + s0, sl)]],
                                     gb.at[pl.ds(s0, sl)], sem)

            def wait(gb, sem):
                pltpu.make_async_copy(tab.at[pl.ds(0, ns)],
                                      gb.at[pl.ds(0, ns)], sem).wait()

            def compute(c, gb):
                for i in range(chi):
                    it = c * chi + i
                    sb = it * 32
                    a0 = jnp.zeros((16,), _f32)
                    a1 = jnp.zeros((16,), _f32)
                    b0 = jnp.zeros((16,), _f32)
                    b1 = jnp.zeros((16,), _f32)
                    for s in range(32):
                        si = i * 32 + s
                        cv = plsc.load_gather(
                            cb, [jnp.full((16,), sb + s, _i32)])
                        if s % 2 == 0:
                            a0 = a0 + cv * gb[si, pl.ds(0, 16)]
                            a1 = a1 + cv * gb[si, pl.ds(16, 16)]
                        else:
                            b0 = b0 + cv * gb[si, pl.ds(0, 16)]
                            b1 = b1 + cv * gb[si, pl.ds(16, 16)]
                    ob[pl.ds(sb, 16)] = a0 + b0
                    ob[pl.ds(sb + 16, 16)] = a1 + b1

            fire(0, g0, sem0)

            def pair(g, carry):
                c = g * 2
                wait(g0, sem0)
                fire(c + 1, g1, sem1)
                compute(c, g0)
                wait(g1, sem1)

                @pl.when(c + 2 < NCH)
                def _():
                    fire(c + 2, g0, sem0)
                compute(c + 1, g1)
                return carry

            lax.fori_loop(0, NCH // 2, pair, 0)
            pltpu.sync_copy(ob.at[pl.ds(0, slab)], out.at[pl.ds(base, slab)])

        side(rows_v, coef_v, tab_a, out_v, ITV, CHV,
             pav if use_plut else None)
        side(rows_a, coef_a, tab_v, out_a, ITA, CHA,
             pvv if use_plut else None)

    return pl.kernel(body, out_type=out_type, mesh=_mesh(),
                     scratch_types=scratch, compiler_params=_SC_PARAMS)


def _make_valproj(n_rows, rb):
    assert n_rows % rb == 0
    grid = (_NL, n_rows // rb)

    def body(x_ref, w_ref, b_ref, o_ref):
        o_ref[...] = (jnp.dot(x_ref[...], w_ref[0],
                              preferred_element_type=_f32) + b_ref[0])[None]

    return pl.pallas_call(
        body,
        grid=grid,
        in_specs=[
            pl.BlockSpec((rb, _D), lambda l, r: (r, 0)),
            pl.BlockSpec((1, _D, _D), lambda l, r: (l, 0, 0)),
            pl.BlockSpec((1, 1, _D), lambda l, r: (l, 0, 0)),
        ],
        out_specs=pl.BlockSpec((1, rb, _D), lambda l, r: (l, r, 0)),
        out_shape=jax.ShapeDtypeStruct((_NL, n_rows, _D), _f32),
    )


def _make_prep(n, k_per_b, lval):
    def body(tgt, pos, ref128, w4, b4, mm, llf, lsi, hvec,
             i0o, i1o, c0o, c1o):
        q = tgt[...] + pos[...]
        lg = jnp.dot(q, w4[...], preferred_element_type=_f32) + b4[...]
        brow = (lax.broadcasted_iota(_i32, (n, 128), 0) >= k_per_b
                ).astype(_i32)
        ref = ref128[...]
        ll = llf[...]
        st = lsi[...]
        hv = hvec[...]
        base = brow * lval + st
        for lid in range(_NL):
            off = lg[:, lid * 256:lid * 256 + 128]
            e = jnp.exp(lg[:, lid * 256 + 128:lid * 256 + 256])
            aw = e / jnp.dot(e, mm[...], preferred_element_type=_f32)
            x = ref * ll + off - 0.5
            x0 = jnp.floor(x)
            w = x - x0
            i0 = jnp.clip(x0, 0.0, ll - 1.0).astype(_i32)
            i1 = jnp.clip(x0 + 1.0, 0.0, ll - 1.0).astype(_i32)
            i0o[lid] = (base + i0) * _H + hv
            i1o[lid] = (base + i1) * _H + hv
            c0o[lid] = aw * (1.0 - w)
            c1o[lid] = aw * w

    return pl.pallas_call(
        body,
        out_shape=[
            jax.ShapeDtypeStruct((_NL, n, 128), _i32),
            jax.ShapeDtypeStruct((_NL, n, 128), _i32),
            jax.ShapeDtypeStruct((_NL, n, 128), _f32),
            jax.ShapeDtypeStruct((_NL, n, 128), _f32),
        ],
    )


def _ln(x, g, b):
    m = jnp.mean(x, -1, keepdims=True)
    v = jnp.mean((x - m) * (x - m), -1, keepdims=True)
    return (x - m) * lax.rsqrt(v + 1e-5) * g + b


def _make_layer(n, with_corr):
    def body(att, tgt, wout, bout, g1, b1, wf1, bf1, wf2, bf2, g2, b2,
             *rest):
        if with_corr:
            wvn, bvn, xo, co = rest
        else:
            (xo,) = rest
        t = jnp.dot(att[...], wout[...], preferred_element_type=_f32) \
            + bout[...]
        h1 = _ln(tgt[...] + t, g1[...], b1[...])
        f = jnp.dot(
            jnp.maximum(
                jnp.dot(h1, wf1[...], preferred_element_type=_f32)
                + bf1[...], 0.0),
            wf2[...], preferred_element_type=_f32) + bf2[...]
        x = _ln(h1 + f, g2[...], b2[...])
        xo[...] = x
        if with_corr:
            co[...] = jnp.dot(x, wvn[...], preferred_element_type=_f32) \
                + bvn[...]

    out_shape = [jax.ShapeDtypeStruct((n, _D), _f32)]
    if with_corr:
        out_shape.append(jax.ShapeDtypeStruct((n, _D), _f32))
    return pl.pallas_call(body, out_shape=out_shape)


_LANE = np.arange(128)
_L_OF_LANE = (_LANE % 16) // _P
_H_OF_LANE = _LANE // 16


def _lane_consts(shapes):
    ll = np.array(shapes, np.float32)[_L_OF_LANE][None]
    st = np.concatenate([[0], np.cumsum(shapes)[:-1]]).astype(np.int32)
    st = st[_L_OF_LANE][None]
    return jnp.asarray(ll), jnp.asarray(st)


def _ref_points(shapes, valid_ratios):
    refs = []
    for l, ll in enumerate(shapes):
        r = (jnp.arange(ll, dtype=_f32) + 0.5)
        r = r[None, :] / (valid_ratios[:, l:l + 1] * ll)
        refs.append(r)
    ref = jnp.concatenate(refs, 1)
    return ref[:, :, None] * valid_ratios[:, None, :]


def _to_flat(a0, a1):
    nl, n, _ = a0.shape
    a = a0.reshape(nl, n, _H, 16)
    b = a1.reshape(nl, n, _H, 16)
    return jnp.concatenate([a, b], axis=-1).reshape(nl, n * _H * 32)


_K = {}


def _build():
    if _K:
        return
    _K["gather0"] = _make_row_gather(
        [_B * _KV, _B * _KV, _B * _KA, _B * _KA], 40)
    _K["finalg"] = _make_row_gather([_B * (_LV + _LA)], 40)
    _K["deform0"] = _make_deform(False)
    _K["deform1"] = _make_deform(True)
    _K["valproj_v"] = _make_valproj(_NVr, 512)
    _K["valproj_a"] = _make_valproj(_NAr, 256)
    _K["prep_v"] = _make_prep(_B * _KV, _KV, _LA)
    _K["prep_a"] = _make_prep(_B * _KA, _KA, _LV)
    _K["layer_v_c"] = _make_layer(_B * _KV, True)
    _K["layer_v"] = _make_layer(_B * _KV, False)
    _K["layer_a_c"] = _make_layer(_B * _KA, True)
    _K["layer_a"] = _make_layer(_B * _KA, False)


def kernel(video_src, video_pos, video_topk, video_valid_ratios,
           audio_src, audio_pos, audio_topk, audio_valid_ratios,
           W_off, b_off, W_attn, b_attn, W_val, b_val, W_out, b_out,
           ln1_g, ln1_b, ln2_g, ln2_b, W_ff1, b_ff1, W_ff2, b_ff2):
    _build()
    bi = jnp.arange(_B, dtype=_i32)[:, None]
    topk_v = video_topk.astype(_i32)
    topk_a = audio_topk.astype(_i32)
    gidx_v = (bi * _LV + topk_v).reshape(-1)
    gidx_a = (bi * _LA + topk_a).reshape(-1)
    srcv = video_src.reshape(_B * _LV, _D)
    posv = video_pos.reshape(_B * _LV, _D)
    srca = audio_src.reshape(_B * _LA, _D)
    posa = audio_pos.reshape(_B * _LA, _D)

    tgt_v, pos_v_t, tgt_a, pos_a_t = _K["gather0"](
        srcv, posv, srca, posa, gidx_v, gidx_v, gidx_a, gidx_a)

    srcv_p = jnp.pad(srcv, ((0, _NVr - _B * _LV), (0, 0)))
    srca_p = jnp.pad(srca, ((0, _NAr - _B * _LA), (0, 0)))
    vbase_v = _K["valproj_v"](srcv_p, W_val[:, 1],
                              b_val[:, 1].reshape(_NL, 1, _D))
    vbase_a = _K["valproj_a"](srca_p, W_val[:, 0],
                              b_val[:, 0].reshape(_NL, 1, _D))

    ref_v = _ref_points(VSHAPES, video_valid_ratios)
    ref_a = _ref_points(ASHAPES, audio_valid_ratios)
    ref_v_t = ref_v[bi, topk_v].reshape(_B * _KV, _L)
    ref_a_t = ref_a[bi, topk_a].reshape(_B * _KA, _L)
    ref128_v = jnp.tile(jnp.repeat(ref_v_t, _P, axis=1), (1, _H))
    ref128_a = jnp.tile(jnp.repeat(ref_a_t, _P, axis=1), (1, _H))

    mm = jnp.asarray(np.kron(np.eye(_H, dtype=np.float32),
                             np.ones((16, 16), np.float32)))
    llf_a, lsi_a = _lane_consts(ASHAPES)
    llf_v, lsi_v = _lane_consts(VSHAPES)
    hvec = jnp.asarray(_H_OF_LANE.astype(np.int32))[None]

    def _w4(m):
        return (jnp.concatenate([W_off[0, m], W_attn[0, m],
                                 W_off[1, m], W_attn[1, m]], axis=1),
                jnp.concatenate([b_off[0, m], b_attn[0, m],
                                 b_off[1, m], b_attn[1, m]])[None])

    w4v, b4v = _w4(0)
    w4a, b4a = _w4(1)
    i0v, i1v, c0v, c1v = _K["prep_v"](tgt_v, pos_v_t, ref128_v, w4v, b4v,
                                      mm, llf_a, lsi_a, hvec)
    i0a, i1a, c0a, c1a = _K["prep_a"](tgt_a, pos_a_t, ref128_a, w4a, b4a,
                                      mm, llf_v, lsi_v, hvec)
    rows_v = _to_flat(i0v, i1v)
    coef_v = _to_flat(c0v, c1v)
    rows_a = _to_flat(i0a, i1a)
    coef_a = _to_flat(c0a, c1a)

    rankv = jnp.broadcast_to(jnp.arange(_KV, dtype=_i32)[None], (_B, _KV))
    ranka = jnp.broadcast_to(jnp.arange(_KA, dtype=_i32)[None], (_B, _KA))
    corrpos_v = (bi * _KV + rankv).reshape(-1)
    corrpos_a = (bi * _KA + ranka).reshape(-1)
    plut_v1 = jnp.arange(_NVr, dtype=_i32).at[gidx_v].set(_NVr + corrpos_v)
    plut_a1 = jnp.arange(_NAr, dtype=_i32).at[gidx_a].set(_NAr + corrpos_a)

    tab_a0 = vbase_a[0].reshape(_NAr * _H, _DH)
    tab_v0 = vbase_v[0].reshape(_NVr * _H, _DH)
    av0, aa0 = _K["deform0"](tab_a0, tab_v0, rows_v[0], coef_v[0],
                             rows_a[0], coef_a[0])
    x_v0, corr_v0 = _K["layer_v_c"](
        av0.reshape(_B * _KV, _D), tgt_v, W_out[0, 0],
        b_out[0, 0][None], ln1_g[0, 0][None], ln1_b[0, 0][None],
        W_ff1[0, 0], b_ff1[0, 0][None], W_ff2[0, 0], b_ff2[0, 0][None],
        ln2_g[0, 0][None], ln2_b[0, 0][None], W_val[1, 1],
        b_val[1, 1][None])
    x_a0, corr_a0 = _K["layer_a_c"](
        aa0.reshape(_B * _KA, _D), tgt_a, W_out[0, 1],
        b_out[0, 1][None], ln1_g[0, 1][None], ln1_b[0, 1][None],
        W_ff1[0, 1], b_ff1[0, 1][None], W_ff2[0, 1], b_ff2[0, 1][None],
        ln2_g[0, 1][None], ln2_b[0, 1][None], W_val[1, 0],
        b_val[1, 0][None])

    tab_v1 = jnp.concatenate([vbase_v[1].reshape(_NVr * _H, _DH),
                              corr_v0.reshape(_B * _KV * _H, _DH)], 0)
    tab_a1 = jnp.concatenate([vbase_a[1].reshape(_NAr * _H, _DH),
                              corr_a0.reshape(_B * _KA * _H, _DH)], 0)
    av1, aa1 = _K["deform1"](tab_a1, tab_v1, rows_v[1], coef_v[1],
                             rows_a[1], coef_a[1], plut_a1, plut_v1)
    (x_v1,) = _K["layer_v"](
        av1.reshape(_B * _KV, _D), tgt_v, W_out[1, 0],
        b_out[1, 0][None], ln1_g[1, 0][None], ln1_b[1, 0][None],
        W_ff1[1, 0], b_ff1[1, 0][None], W_ff2[1, 0], b_ff2[1, 0][None],
        ln2_g[1, 0][None], ln2_b[1, 0][None])
    (x_a1,) = _K["layer_a"](
        aa1.reshape(_B * _KA, _D), tgt_a, W_out[1, 1],
        b_out[1, 1][None], ln1_g[1, 1][None], ln1_b[1, 1][None],
        W_ff1[1, 1], b_ff1[1, 1][None], W_ff2[1, 1], b_ff2[1, 1][None],
        ln2_g[1, 1][None], ln2_b[1, 1][None])

    ftab = jnp.concatenate([srcv, x_v1, srca, x_a1], 0)
    nv, na = _B * _LV, _B * _LA
    lv = jnp.arange(nv, dtype=_i32).at[gidx_v].set(nv + corrpos_v)
    la = (nv + _B * _KV + jnp.arange(na, dtype=_i32)).at[gidx_a].set(
        nv + _B * _KV + na + corrpos_a)
    lut_full = jnp.concatenate([lv[:_LV], la[:_LA],
                                lv[_LV:], la[_LA:]], 0)
    (out_flat,) = _K["finalg"](ftab, lut_full)
    return out_flat.reshape(_B, _LV + _LA, _D)
